# baseline (speedup 1.0000x reference)
import jax
import jax.numpy as jnp
from jax.experimental import pallas as pl
from jax.experimental.pallas import tpu as pltpu

_DEPTH = 4
_BLOCK = 256
_QUEUES = 4


def _gcn_body(fn_ref, nf_ref, x_ref, w1_ref, b1_ref, w2_ref, b2_ref,
              out_ref, buf_ref, bblk_ref, s1_ref, s2_ref, sem_ref):
    m, _ = fn_ref.shape
    n, _ = nf_ref.shape
    nchunk1 = m // _BLOCK
    nchunk2 = n // _BLOCK
    total = nchunk1 + nchunk2

    sub = _BLOCK // _QUEUES

    def copy_in(c, slot):
        c1i = jnp.minimum(c, nchunk1 - 1)
        c2i = jnp.maximum(c - nchunk1, 0)

        def _fn():
            for q in range(_QUEUES):
                pltpu.make_async_copy(
                    fn_ref.at[pl.ds(c1i * _BLOCK + q * sub, sub), :],
                    buf_ref.at[slot, pl.ds(q * sub, sub), :],
                    sem_ref.at[slot, q]).start()

        def _nf():
            for q in range(_QUEUES):
                pltpu.make_async_copy(
                    nf_ref.at[pl.ds(c2i * _BLOCK + q * sub, sub), :],
                    buf_ref.at[slot, pl.ds(q * sub, sub), :],
                    sem_ref.at[slot, q]).start()

        jax.lax.cond(c < nchunk1, _fn, _nf)

    def wait(slot):
        for q in range(_QUEUES):
            pltpu.make_async_copy(
                fn_ref.at[pl.ds(0, sub), :],
                buf_ref.at[slot, pl.ds(q * sub, sub), :],
                sem_ref.at[slot, q]).wait()

    for c in range(_DEPTH):
        copy_in(c, c)
    s1_ref[...] = jnp.dot(x_ref[...], w1_ref[...],
                          preferred_element_type=jnp.float32
                          ).astype(jnp.bfloat16)
    w2b = w2_ref[...].astype(jnp.bfloat16)

    def body1(c, carry):
        slot = jax.lax.rem(c, _DEPTH)
        wait(slot)
        bblk_ref[...] = buf_ref[slot].astype(jnp.bfloat16)
        acc = jnp.dot(bblk_ref[...], s1_ref[...],
                      preferred_element_type=jnp.float32)
        hc = jnp.maximum(acc + b1_ref[...], 0.0).astype(jnp.bfloat16)
        s2_ref[pl.ds(c * _BLOCK, _BLOCK), :] = jnp.dot(
            hc, w2b, preferred_element_type=jnp.float32).astype(jnp.bfloat16)
        nxt = c + _DEPTH

        @pl.when(nxt < total)
        def _():
            copy_in(nxt, slot)
        return carry

    jax.lax.fori_loop(0, nchunk1, body1, 0)

    def body2(c, carry):
        slot = jax.lax.rem(c, _DEPTH)
        wait(slot)
        bblk_ref[...] = buf_ref[slot].astype(jnp.bfloat16)
        acc = jnp.dot(bblk_ref[...], s2_ref[...],
                      preferred_element_type=jnp.float32)
        out_ref[pl.ds((c - nchunk1) * _BLOCK, _BLOCK), :] = acc + b2_ref[...]
        nxt = c + _DEPTH

        @pl.when(nxt < total)
        def _():
            copy_in(nxt, slot)
        return carry

    jax.lax.fori_loop(nchunk1, total, body2, 0)


def kernel(x, NF, FN, W1, b1, W2, b2):
    m, k = FN.shape
    n, _ = NF.shape
    kf, f = x.shape
    c1 = W1.shape[1]
    c2 = W2.shape[1]
    return pl.pallas_call(
        _gcn_body,
        in_specs=[
            pl.BlockSpec(memory_space=pltpu.MemorySpace.HBM),
            pl.BlockSpec(memory_space=pltpu.MemorySpace.HBM),
            pl.BlockSpec((kf, f), lambda: (0, 0)),
            pl.BlockSpec((f, c1), lambda: (0, 0)),
            pl.BlockSpec((1, c1), lambda: (0, 0)),
            pl.BlockSpec((c1, c2), lambda: (0, 0)),
            pl.BlockSpec((1, c2), lambda: (0, 0)),
        ],
        out_specs=pl.BlockSpec((n, c2), lambda: (0, 0)),
        out_shape=jax.ShapeDtypeStruct((n, c2), jnp.float32),
        scratch_shapes=[
            pltpu.VMEM((_DEPTH, _BLOCK, k), jnp.float32),
            pltpu.VMEM((_BLOCK, k), jnp.bfloat16),
            pltpu.VMEM((kf, c1), jnp.bfloat16),
            pltpu.VMEM((m, c2), jnp.bfloat16),
            pltpu.SemaphoreType.DMA((_DEPTH, _QUEUES)),
        ],
    )(FN, NF, x, W1, b1.reshape(1, -1), W2, b2.reshape(1, -1))


# speedup vs baseline: 1.0048x; 1.0048x over previous
import jax
import jax.numpy as jnp
from jax.experimental import pallas as pl
from jax.experimental.pallas import tpu as pltpu

_DEPTH = 4
_BLOCK = 256
_QUEUES = 4


def _gcn_body(fn_ref, nf_ref, x_ref, w1_ref, b1_ref, w2_ref, b2_ref,
              out_ref, buf_ref, s1_ref, s2_ref, sem_ref):
    m, _ = fn_ref.shape
    n, _ = nf_ref.shape
    nchunk1 = m // _BLOCK
    nchunk2 = n // _BLOCK
    total = nchunk1 + nchunk2

    sub = _BLOCK // _QUEUES

    def copy_in(c, slot):
        c1i = jnp.minimum(c, nchunk1 - 1)
        c2i = jnp.maximum(c - nchunk1, 0)

        def _fn():
            for q in range(_QUEUES):
                pltpu.make_async_copy(
                    fn_ref.at[pl.ds(c1i * _BLOCK + q * sub, sub), :],
                    buf_ref.at[slot, pl.ds(q * sub, sub), :],
                    sem_ref.at[slot, q]).start()

        def _nf():
            for q in range(_QUEUES):
                pltpu.make_async_copy(
                    nf_ref.at[pl.ds(c2i * _BLOCK + q * sub, sub), :],
                    buf_ref.at[slot, pl.ds(q * sub, sub), :],
                    sem_ref.at[slot, q]).start()

        jax.lax.cond(c < nchunk1, _fn, _nf)

    def wait(slot):
        for q in range(_QUEUES):
            pltpu.make_async_copy(
                fn_ref.at[pl.ds(0, sub), :],
                buf_ref.at[slot, pl.ds(q * sub, sub), :],
                sem_ref.at[slot, q]).wait()

    for c in range(_DEPTH):
        copy_in(c, c)
    s1_ref[...] = jnp.dot(x_ref[...], w1_ref[...],
                          preferred_element_type=jnp.float32
                          ).astype(jnp.bfloat16)
    w2b = w2_ref[...].astype(jnp.bfloat16)

    def body1(c, carry):
        slot = jax.lax.rem(c, _DEPTH)
        wait(slot)
        blk = buf_ref[slot].astype(jnp.bfloat16)
        acc = jnp.dot(blk, s1_ref[...], preferred_element_type=jnp.float32)
        hc = jnp.maximum(acc + b1_ref[...], 0.0).astype(jnp.bfloat16)
        s2_ref[pl.ds(c * _BLOCK, _BLOCK), :] = jnp.dot(
            hc, w2b, preferred_element_type=jnp.float32).astype(jnp.bfloat16)
        nxt = c + _DEPTH

        @pl.when(nxt < total)
        def _():
            copy_in(nxt, slot)
        return carry

    jax.lax.fori_loop(0, nchunk1, body1, 0)

    def body2(c, carry):
        slot = jax.lax.rem(c, _DEPTH)
        wait(slot)
        blk = buf_ref[slot].astype(jnp.bfloat16)
        acc = jnp.dot(blk, s2_ref[...], preferred_element_type=jnp.float32)
        out_ref[pl.ds((c - nchunk1) * _BLOCK, _BLOCK), :] = acc + b2_ref[...]
        nxt = c + _DEPTH

        @pl.when(nxt < total)
        def _():
            copy_in(nxt, slot)
        return carry

    jax.lax.fori_loop(nchunk1, total, body2, 0)


def kernel(x, NF, FN, W1, b1, W2, b2):
    m, k = FN.shape
    n, _ = NF.shape
    kf, f = x.shape
    c1 = W1.shape[1]
    c2 = W2.shape[1]
    return pl.pallas_call(
        _gcn_body,
        in_specs=[
            pl.BlockSpec(memory_space=pltpu.MemorySpace.HBM),
            pl.BlockSpec(memory_space=pltpu.MemorySpace.HBM),
            pl.BlockSpec((kf, f), lambda: (0, 0)),
            pl.BlockSpec((f, c1), lambda: (0, 0)),
            pl.BlockSpec((1, c1), lambda: (0, 0)),
            pl.BlockSpec((c1, c2), lambda: (0, 0)),
            pl.BlockSpec((1, c2), lambda: (0, 0)),
        ],
        out_specs=pl.BlockSpec((n, c2), lambda: (0, 0)),
        out_shape=jax.ShapeDtypeStruct((n, c2), jnp.float32),
        scratch_shapes=[
            pltpu.VMEM((_DEPTH, _BLOCK, k), jnp.float32),
            pltpu.VMEM((kf, c1), jnp.bfloat16),
            pltpu.VMEM((m, c2), jnp.bfloat16),
            pltpu.SemaphoreType.DMA((_DEPTH, _QUEUES)),
        ],
    )(FN, NF, x, W1, b1.reshape(1, -1), W2, b2.reshape(1, -1))
